# baseline (speedup 1.0000x reference)
import jax
import jax.numpy as jnp
from jax.experimental import pallas as pl
from jax.experimental.pallas import tpu as pltpu

_B = 4000
_NB = 250
_Q = 64
_INF = float("inf")
_IMAX = 2**31 - 1


def _lex_cmpswap(va, ia, vb, ib):
    take_a = (va < vb) | ((va == vb) & (ia < ib))
    lo_v = jnp.where(take_a, va, vb)
    lo_i = jnp.where(take_a, ia, ib)
    hi_v = jnp.where(take_a, vb, va)
    hi_i = jnp.where(take_a, ib, ia)
    return lo_v, lo_i, hi_v, hi_i


def _merge4(rv, ri, bv, bi):
    l = []
    for i in range(4):
        lo_v, lo_i, _, _ = _lex_cmpswap(
            rv[i:i + 1], ri[i:i + 1], bv[3 - i:4 - i], bi[3 - i:4 - i])
        l.append((lo_v, lo_i))
    v0, i0, v2, i2 = _lex_cmpswap(l[0][0], l[0][1], l[2][0], l[2][1])
    v1, i1, v3, i3 = _lex_cmpswap(l[1][0], l[1][1], l[3][0], l[3][1])
    v0, i0, v1, i1 = _lex_cmpswap(v0, i0, v1, i1)
    v2, i2, v3, i3 = _lex_cmpswap(v2, i2, v3, i3)
    return (jnp.concatenate([v0, v1, v2, v3], axis=0),
            jnp.concatenate([i0, i1, i2, i3], axis=0))


def _body(m2q_ref, qsq_ref, kb_ref, vals_ref, idx_ref, rs_ref, ri_ref, s_ref):
    b = pl.program_id(0)

    @pl.when(b == 0)
    def _init():
        rs_ref[...] = jnp.full((4, _Q), _INF, jnp.float32)
        ri_ref[...] = jnp.zeros((4, _Q), jnp.int32)

    kb = kb_ref[...]
    m2q = m2q_ref[...]
    ksq = jnp.sum(kb * kb, axis=1, keepdims=True)
    s = jax.lax.dot_general(
        kb, m2q, (((1,), (1,)), ((), ())),
        preferred_element_type=jnp.float32) + ksq
    s_ref[...] = s

    rowid = jax.lax.broadcasted_iota(jnp.int32, (_B, _Q), 0)
    base = b * _B
    ext_v, ext_i = [], []
    for r in range(4):
        sc = s_ref[...]
        m = jnp.min(sc, axis=0, keepdims=True)
        cand = jnp.where(sc == m, rowid, _IMAX)
        am = jnp.min(cand, axis=0, keepdims=True)
        ext_v.append(m)
        ext_i.append(am + base)
        if r < 3:
            s_ref[...] = jnp.where(rowid == am, _INF, sc)

    bv = jnp.concatenate(ext_v, axis=0)
    bi = jnp.concatenate(ext_i, axis=0)
    nv, ni = _merge4(rs_ref[...], ri_ref[...], bv, bi)
    rs_ref[...] = nv
    ri_ref[...] = ni

    @pl.when(b == _NB - 1)
    def _fin():
        d2 = rs_ref[...] + qsq_ref[...]
        vals_ref[...] = jnp.sqrt(jnp.maximum(d2, 1e-12))
        idx_ref[...] = ri_ref[...]


def kernel(queries, keys, k):
    m2q = -2.0 * queries
    qsq = jnp.sum(queries * queries, axis=1)[None, :]
    vals, idx = pl.pallas_call(
        _body,
        grid=(_NB,),
        in_specs=[
            pl.BlockSpec((_Q, 16), lambda b: (0, 0)),
            pl.BlockSpec((1, _Q), lambda b: (0, 0)),
            pl.BlockSpec((_B, 16), lambda b: (b, 0)),
        ],
        out_specs=[
            pl.BlockSpec((4, _Q), lambda b: (0, 0)),
            pl.BlockSpec((4, _Q), lambda b: (0, 0)),
        ],
        out_shape=[
            jax.ShapeDtypeStruct((4, _Q), jnp.float32),
            jax.ShapeDtypeStruct((4, _Q), jnp.int32),
        ],
        scratch_shapes=[
            pltpu.VMEM((4, _Q), jnp.float32),
            pltpu.VMEM((4, _Q), jnp.int32),
            pltpu.VMEM((_B, _Q), jnp.float32),
        ],
    )(m2q, qsq, keys)
    top_dist = vals.T
    indices = idx.T + (jnp.asarray(k, jnp.int32) - 4)
    return top_dist, indices


# speedup vs baseline: 1.4062x; 1.4062x over previous
import jax
import jax.numpy as jnp
from jax.experimental import pallas as pl
from jax.experimental.pallas import tpu as pltpu

_B = 8000
_NB = 125
_Q = 64
_INF = float("inf")
_IMAX = 2**31 - 1


def _lex_cmpswap(va, ia, vb, ib):
    take_a = (va < vb) | ((va == vb) & (ia < ib))
    lo_v = jnp.where(take_a, va, vb)
    lo_i = jnp.where(take_a, ia, ib)
    hi_v = jnp.where(take_a, vb, va)
    hi_i = jnp.where(take_a, ib, ia)
    return lo_v, lo_i, hi_v, hi_i


def _merge4(rv, ri, bv, bi):
    l = []
    for i in range(4):
        lo_v, lo_i, _, _ = _lex_cmpswap(
            rv[i:i + 1], ri[i:i + 1], bv[3 - i:4 - i], bi[3 - i:4 - i])
        l.append((lo_v, lo_i))
    v0, i0, v2, i2 = _lex_cmpswap(l[0][0], l[0][1], l[2][0], l[2][1])
    v1, i1, v3, i3 = _lex_cmpswap(l[1][0], l[1][1], l[3][0], l[3][1])
    v0, i0, v1, i1 = _lex_cmpswap(v0, i0, v1, i1)
    v2, i2, v3, i3 = _lex_cmpswap(v2, i2, v3, i3)
    return (jnp.concatenate([v0, v1, v2, v3], axis=0),
            jnp.concatenate([i0, i1, i2, i3], axis=0))


def _body(m2q_ref, qsq_ref, kb_ref, vals_ref, idx_ref,
          rs_ref, ri_ref, s_ref, mrow_ref, ev_ref, ei_ref):
    b = pl.program_id(0)

    @pl.when(b == 0)
    def _init():
        rs_ref[...] = jnp.full((4, _Q), _INF, jnp.float32)
        ri_ref[...] = jnp.full((4, _Q), _IMAX, jnp.int32)

    kb = kb_ref[...]
    m2q = m2q_ref[...]
    ksq = jnp.sum(kb * kb, axis=1, keepdims=True)
    s = jax.lax.dot_general(
        kb, m2q, (((1,), (1,)), ((), ())),
        preferred_element_type=jnp.float32) + ksq
    m0 = jnp.min(s, axis=0, keepdims=True)

    @pl.when(jnp.any(m0 < rs_ref[3:4, :]))
    def _extract():
        s_ref[...] = s
        mrow_ref[...] = m0
        ev_ref[...] = jnp.full((4, _Q), _INF, jnp.float32)
        ei_ref[...] = jnp.full((4, _Q), _IMAX, jnp.int32)
        rowid = jax.lax.broadcasted_iota(jnp.int32, (_B, _Q), 0)
        base = b * _B
        for r in range(4):
            @pl.when(jnp.any(mrow_ref[...] < rs_ref[3:4, :]))
            def _round(r=r):
                sc = s_ref[...]
                m = mrow_ref[...]
                cand = jnp.where(sc == m, rowid, _IMAX)
                am = jnp.min(cand, axis=0, keepdims=True)
                ev_ref[r:r + 1, :] = m
                ei_ref[r:r + 1, :] = am + base
                if r < 3:
                    masked = jnp.where(rowid == am, _INF, sc)
                    s_ref[...] = masked
                    mrow_ref[...] = jnp.min(masked, axis=0, keepdims=True)
        nv, ni = _merge4(rs_ref[...], ri_ref[...], ev_ref[...], ei_ref[...])
        rs_ref[...] = nv
        ri_ref[...] = ni

    @pl.when(b == _NB - 1)
    def _fin():
        d2 = rs_ref[...] + qsq_ref[...]
        vals_ref[...] = jnp.sqrt(jnp.maximum(d2, 1e-12))
        idx_ref[...] = ri_ref[...]


def kernel(queries, keys, k):
    m2q = -2.0 * queries
    qsq = jnp.sum(queries * queries, axis=1)[None, :]
    vals, idx = pl.pallas_call(
        _body,
        grid=(_NB,),
        in_specs=[
            pl.BlockSpec((_Q, 16), lambda b: (0, 0)),
            pl.BlockSpec((1, _Q), lambda b: (0, 0)),
            pl.BlockSpec((_B, 16), lambda b: (b, 0)),
        ],
        out_specs=[
            pl.BlockSpec((4, _Q), lambda b: (0, 0)),
            pl.BlockSpec((4, _Q), lambda b: (0, 0)),
        ],
        out_shape=[
            jax.ShapeDtypeStruct((4, _Q), jnp.float32),
            jax.ShapeDtypeStruct((4, _Q), jnp.int32),
        ],
        scratch_shapes=[
            pltpu.VMEM((4, _Q), jnp.float32),
            pltpu.VMEM((4, _Q), jnp.int32),
            pltpu.VMEM((_B, _Q), jnp.float32),
            pltpu.VMEM((1, _Q), jnp.float32),
            pltpu.VMEM((4, _Q), jnp.float32),
            pltpu.VMEM((4, _Q), jnp.int32),
        ],
    )(m2q, qsq, keys)
    top_dist = vals.T
    indices = idx.T + (jnp.asarray(k, jnp.int32) - 4)
    return top_dist, indices
